# Initial kernel scaffold; baseline (speedup 1.0000x reference)
import jax
import jax.numpy as jnp
from jax.experimental import pallas as pl


def kernel(x, edge_index, batch, emb, W_root, W_rel, b_rel, gamma, beta, W_ih, W_hh, b_ih, b_hh, fc_W, fc_b):
    raise NotImplementedError("write your pallas kernel here")


# speedup vs baseline: 60.4515x; 60.4515x over previous
import functools

import jax
import jax.numpy as jnp
from jax import lax
from jax.experimental import pallas as pl
from jax.experimental.pallas import tpu as pltpu
from jax.experimental.pallas import tpu_sc as plsc

N = 10000
E = 320000
B = 100
D = 128
H = 128
V = 21000

NC = 2
NS = 16
NW = NC * NS
CH = 128

XCHUNKS = 3 * NW
XP = XCHUNKS * CH

ECHUNKS = E // CH
EFULL = ECHUNKS // NW
EREM = ECHUNKS - EFULL * NW

ROWS_PER_TILE = 632
NPAD = NS * ROWS_PER_TILE
_ZW_CHUNKS = [(0, 128), (128, 128), (256, 128), (384, 128), (512, 120)]

BP = 104
NULLROW = N
BATCH_PAD = (N + CH - 1) // CH * CH


def _emb_gather_kernel(emb_hbm, x_hbm, out_hbm, idx_v, rows_v, sem):
    wid = lax.axis_index("s") * NC + lax.axis_index("c")
    for j in range(3):
        base = (wid * 3 + j) * CH
        pltpu.sync_copy(x_hbm.at[pl.ds(base, CH)], idx_v)
        pltpu.async_copy(emb_hbm.at[idx_v], rows_v, sem).wait()
        pltpu.sync_copy(rows_v, out_hbm.at[pl.ds(base, CH)])


def _edge_aggr_kernel(feat_hbm, src_hbm, dst_hbm, z_hbm, out_hbm,
                      src_v, dst_v, rows_v, aggr_sh, sem):
    cid = lax.axis_index("c")
    sid = lax.axis_index("s")
    wid = sid * NC + cid

    r0 = sid * ROWS_PER_TILE
    pltpu.sync_copy(z_hbm, rows_v)
    for off, sz in _ZW_CHUNKS:
        pltpu.sync_copy(rows_v.at[pl.ds(0, sz)],
                        aggr_sh.at[pl.ds(r0 + off, sz)])
    plsc.subcore_barrier()

    def chunk(c):
        base = c * CH
        pltpu.sync_copy(src_hbm.at[pl.ds(base, CH)], src_v)
        pltpu.sync_copy(dst_hbm.at[pl.ds(base, CH)], dst_v)
        pltpu.async_copy(feat_hbm.at[src_v], rows_v, sem).wait()
        pltpu.sync_copy(rows_v, aggr_sh.at[dst_v], add=True)

    def body(j, carry):
        chunk(wid + j * NW)
        return carry

    lax.fori_loop(0, EFULL, body, 0)

    @pl.when(wid < EREM)
    def _():
        chunk(wid + EFULL * NW)

    plsc.subcore_barrier()
    for off, sz in _ZW_CHUNKS:
        pltpu.sync_copy(aggr_sh.at[pl.ds(r0 + off, sz)],
                        rows_v.at[pl.ds(0, sz)])
        pltpu.sync_copy(rows_v.at[pl.ds(0, sz)],
                        out_hbm.at[pl.ds(cid * NPAD + r0 + off, sz)])


def _graphconv_kernel(p0_ref, p1_ref, feat_ref, wrel_ref, wroot_ref,
                      brel_ref, h_ref, sum_ref, sq_ref):
    i = pl.program_id(0)
    a = p0_ref[...] + p1_ref[...]
    h = (jnp.dot(a, wrel_ref[...], preferred_element_type=jnp.float32)
         + jnp.dot(feat_ref[...], wroot_ref[...],
                   preferred_element_type=jnp.float32)
         + brel_ref[...])
    h = jnp.maximum(h, 0.0)
    h_ref[...] = h

    @pl.when(i == 0)
    def _():
        sum_ref[...] = jnp.zeros_like(sum_ref)
        sq_ref[...] = jnp.zeros_like(sq_ref)

    sum_ref[...] += jnp.sum(h, axis=0, keepdims=True)
    sq_ref[...] += jnp.sum(h * h, axis=0, keepdims=True)


def _lstm_kernel(h_ref, sum_ref, sq_ref, batch_ref, gamma_ref, beta_ref,
                 wih_ref, whh_ref, bih_ref, bhh_ref, fcw_ref, fcb_ref,
                 out_ref, hbuf, xbuf, hst, cst, cnt_sm):
    batch2d = batch_ref[...]

    def count_body(b, carry):
        cnt_sm[b] = jnp.sum(jnp.where(batch2d < b, 1, 0))
        return carry

    lax.fori_loop(0, B + 5, count_body, 0)

    def max_body(b, m):
        return jnp.maximum(m, cnt_sm[b + 1] - cnt_sm[b])

    tmax = lax.fori_loop(0, B, max_body, 0)

    mean = sum_ref[...] / N
    var = sq_ref[...] / N - mean * mean
    scale = gamma_ref[...] * lax.rsqrt(var + 1e-5)
    shift = beta_ref[...] - mean * scale
    hbuf[pl.ds(0, N), :] = h_ref[...] * scale + shift
    hbuf[pl.ds(N, 8), :] = jnp.zeros((8, H), jnp.float32)

    hst[...] = jnp.zeros((BP, H), jnp.float32)
    cst[...] = jnp.zeros((BP, H), jnp.float32)

    bias = bih_ref[...] + bhh_ref[...]

    def step(t, carry):
        def gather(b, c2):
            s0 = cnt_sm[b]
            nb = cnt_sm[b + 1] - s0
            idx = jnp.where(t + 1 < nb, s0 + t + 1, NULLROW)
            xbuf[pl.ds(b, 1), :] = hbuf[pl.ds(idx, 1), :]
            return c2

        lax.fori_loop(0, BP, gather, 0)

        gates = (jnp.dot(xbuf[...], wih_ref[...],
                         preferred_element_type=jnp.float32)
                 + jnp.dot(hst[...], whh_ref[...],
                           preferred_element_type=jnp.float32)
                 + bias)
        i_g = gates[:, 0:H]
        f_g = gates[:, H:2 * H]
        g_g = gates[:, 2 * H:3 * H]
        o_g = gates[:, 3 * H:4 * H]
        c = (jax.nn.sigmoid(f_g) * cst[...]
             + jax.nn.sigmoid(i_g) * jnp.tanh(g_g))
        hst[...] = jax.nn.sigmoid(o_g) * jnp.tanh(c)
        cst[...] = c
        return carry

    lax.fori_loop(0, tmax - 1, step, 0)

    s = jnp.sum(hst[...] * fcw_ref[...], axis=1, keepdims=True)
    out_ref[...] = jnp.maximum(s + fcb_ref[...], 0.0)


def kernel(x, edge_index, batch, emb, W_root, W_rel, b_rel, gamma, beta,
           W_ih, W_hh, b_ih, b_hh, fc_W, fc_b):
    src = edge_index[0]
    dst = edge_index[1]

    x_pad = jnp.pad(x, (0, XP - N))
    mesh = plsc.VectorSubcoreMesh(core_axis_name="c", subcore_axis_name="s",
                                  num_cores=NC, num_subcores=NS)
    feat_xp = pl.kernel(
        _emb_gather_kernel,
        out_type=jax.ShapeDtypeStruct((XP, D), jnp.float32),
        mesh=mesh,
        scratch_types=[
            pltpu.VMEM((CH,), jnp.int32),
            pltpu.VMEM((CH, D), jnp.float32),
            pltpu.SemaphoreType.DMA,
        ],
    )(emb, x_pad)

    zeros_tile = jnp.zeros((CH, D), jnp.float32)
    aggr2 = pl.kernel(
        _edge_aggr_kernel,
        out_type=jax.ShapeDtypeStruct((NC * NPAD, D), jnp.float32),
        mesh=mesh,
        scratch_types=[
            pltpu.VMEM((CH,), jnp.int32),
            pltpu.VMEM((CH,), jnp.int32),
            pltpu.VMEM((CH, D), jnp.float32),
            pltpu.VMEM_SHARED((NPAD, D), jnp.float32),
            pltpu.SemaphoreType.DMA,
        ],
    )(feat_xp, src, dst, zeros_tile)

    feat = feat_xp[:N]
    p0 = aggr2[:N]
    p1 = aggr2[NPAD:NPAD + N]

    RT = 1000
    NT = N // RT
    h_pre, col_sum, col_sq = pl.pallas_call(
        _graphconv_kernel,
        grid=(NT,),
        in_specs=[
            pl.BlockSpec((RT, D), lambda i: (i, 0)),
            pl.BlockSpec((RT, D), lambda i: (i, 0)),
            pl.BlockSpec((RT, D), lambda i: (i, 0)),
            pl.BlockSpec((D, H), lambda i: (0, 0)),
            pl.BlockSpec((D, H), lambda i: (0, 0)),
            pl.BlockSpec((1, H), lambda i: (0, 0)),
        ],
        out_specs=[
            pl.BlockSpec((RT, H), lambda i: (i, 0)),
            pl.BlockSpec((1, H), lambda i: (0, 0)),
            pl.BlockSpec((1, H), lambda i: (0, 0)),
        ],
        out_shape=[
            jax.ShapeDtypeStruct((N, H), jnp.float32),
            jax.ShapeDtypeStruct((1, H), jnp.float32),
            jax.ShapeDtypeStruct((1, H), jnp.float32),
        ],
    )(p0, p1, feat, W_rel, W_root, b_rel.reshape(1, H))

    batch_pad = jnp.pad(batch, (0, BATCH_PAD - N), constant_values=B)
    out = pl.pallas_call(
        _lstm_kernel,
        out_shape=jax.ShapeDtypeStruct((BP, 1), jnp.float32),
        scratch_shapes=[
            pltpu.VMEM((N + 8, H), jnp.float32),
            pltpu.VMEM((BP, H), jnp.float32),
            pltpu.VMEM((BP, H), jnp.float32),
            pltpu.VMEM((BP, H), jnp.float32),
            pltpu.SMEM((128,), jnp.int32),
        ],
    )(h_pre, col_sum, col_sq, batch_pad.reshape(BATCH_PAD // CH, CH),
      gamma.reshape(1, H), beta.reshape(1, H),
      W_ih.T, W_hh.T, b_ih.reshape(1, 4 * H), b_hh.reshape(1, 4 * H),
      fc_W.reshape(1, H), fc_b.reshape(1, 1))

    return out[:B, 0]


# speedup vs baseline: 96.6851x; 1.5994x over previous
import functools

import jax
import jax.numpy as jnp
from jax import lax
from jax.experimental import pallas as pl
from jax.experimental.pallas import tpu as pltpu
from jax.experimental.pallas import tpu_sc as plsc

N = 10000
E = 320000
B = 100
D = 128
H = 128
V = 21000

NC = 2
NS = 16
NW = NC * NS
CH = 128

XCHUNKS = 3 * NW
XP = XCHUNKS * CH

ECHUNKS = E // CH
EFULL = ECHUNKS // NW
EREM = ECHUNKS - EFULL * NW

ROWS_PER_TILE = 632
NPAD = NS * ROWS_PER_TILE
_ZW_CHUNKS = [(0, 128), (128, 128), (256, 128), (384, 128), (512, 120)]

BP = 104
TB = 16
BATCH_PAD = (N + CH - 1) // CH * CH


def _emb_gather_kernel(emb_hbm, x_hbm, out_hbm,
                       idx0, idx1, rows0, rows1, sem0, sem1):
    wid = lax.axis_index("s") * NC + lax.axis_index("c")
    idx = [idx0, idx1]
    rows = [rows0, rows1]
    sem = [sem0, sem1]
    base = [(wid * 3 + j) * CH for j in range(3)]

    pltpu.sync_copy(x_hbm.at[pl.ds(base[0], CH)], idx[0])
    g = pltpu.async_copy(emb_hbm.at[idx[0]], rows[0], sem[0])
    for j in range(3):
        p = j % 2
        if j + 1 < 3:
            q = (j + 1) % 2
            pltpu.sync_copy(x_hbm.at[pl.ds(base[j + 1], CH)], idx[q])
            gn = pltpu.async_copy(emb_hbm.at[idx[q]], rows[q], sem[q])
        g.wait()
        pltpu.sync_copy(rows[p], out_hbm.at[pl.ds(base[j], CH)])
        if j + 1 < 3:
            g = gn


def _edge_aggr_kernel(feat_hbm, src_hbm, dst_hbm, z_hbm, out_hbm,
                      src0, dst0, src1, dst1, rows0, rows1,
                      aggr_sh, sem0, sem1):
    cid = lax.axis_index("c")
    sid = lax.axis_index("s")
    wid = sid * NC + cid

    r0 = sid * ROWS_PER_TILE
    pltpu.sync_copy(z_hbm, rows0)
    for off, sz in _ZW_CHUNKS:
        pltpu.sync_copy(rows0.at[pl.ds(0, sz)],
                        aggr_sh.at[pl.ds(r0 + off, sz)])
    plsc.subcore_barrier()

    has_rem = wid < EREM
    jlast = EFULL - 1 + jnp.where(has_rem, 1, 0)

    def load(j, s_v, d_v):
        base = (wid + j * NW) * CH
        pltpu.sync_copy(src_hbm.at[pl.ds(base, CH)], s_v)
        pltpu.sync_copy(dst_hbm.at[pl.ds(base, CH)], d_v)

    load(0, src0, dst0)
    pltpu.async_copy(feat_hbm.at[src0], rows0, sem0)

    def drain(s_v, r_v, sem):
        pltpu.make_async_copy(feat_hbm.at[s_v], r_v, sem).wait()

    def body(k, carry):
        load(2 * k + 1, src1, dst1)
        pltpu.async_copy(feat_hbm.at[src1], rows1, sem1)
        drain(src0, rows0, sem0)
        pltpu.sync_copy(rows0, aggr_sh.at[dst0], add=True)
        load(jnp.minimum(2 * k + 2, jlast), src0, dst0)
        pltpu.async_copy(feat_hbm.at[src0], rows0, sem0)
        drain(src1, rows1, sem1)
        pltpu.sync_copy(rows1, aggr_sh.at[dst1], add=True)
        return carry

    lax.fori_loop(0, EFULL // 2, body, 0)

    drain(src0, rows0, sem0)

    @pl.when(has_rem)
    def _():
        pltpu.sync_copy(rows0, aggr_sh.at[dst0], add=True)

    plsc.subcore_barrier()
    for off, sz in _ZW_CHUNKS:
        pltpu.sync_copy(aggr_sh.at[pl.ds(r0 + off, sz)],
                        rows0.at[pl.ds(0, sz)])
        pltpu.sync_copy(rows0.at[pl.ds(0, sz)],
                        out_hbm.at[pl.ds(cid * NPAD + r0 + off, sz)])


def _graphconv_kernel(p0_ref, p1_ref, feat_ref, wrel_ref, wroot_ref,
                      brel_ref, h_ref, sum_ref, sq_ref):
    i = pl.program_id(0)
    a = p0_ref[...] + p1_ref[...]
    h = (jnp.dot(a, wrel_ref[...], preferred_element_type=jnp.float32)
         + jnp.dot(feat_ref[...], wroot_ref[...],
                   preferred_element_type=jnp.float32)
         + brel_ref[...])
    h = jnp.maximum(h, 0.0)
    h_ref[...] = h

    @pl.when(i == 0)
    def _():
        sum_ref[...] = jnp.zeros_like(sum_ref)
        sq_ref[...] = jnp.zeros_like(sq_ref)

    sum_ref[...] += jnp.sum(h, axis=0, keepdims=True)
    sq_ref[...] += jnp.sum(h * h, axis=0, keepdims=True)


def _lstm_kernel(h_ref, sum_ref, sq_ref, batch_ref, gamma_ref, beta_ref,
                 wih_ref, whh_ref, bih_ref, bhh_ref, fcw_ref, fcb_ref,
                 out_ref, hbuf, xblk, nbuf, hst, cst, cnt_sm):
    batch2d = batch_ref[...]

    def count_body(b, carry):
        cnt_sm[b] = jnp.sum(jnp.where(batch2d < b, 1, 0))
        return carry

    lax.fori_loop(0, B + 5, count_body, 0)

    def max_body(b, m):
        return jnp.maximum(m, cnt_sm[b + 1] - cnt_sm[b])

    tmax = lax.fori_loop(0, B, max_body, 0)

    def nb_body(b, carry):
        nb = cnt_sm[b + 1] - cnt_sm[b]
        nbuf[pl.ds(b, 1), :] = jnp.full((1, 1), nb, jnp.int32)
        return carry

    lax.fori_loop(0, BP, nb_body, 0)

    mean = sum_ref[...] / N
    var = sq_ref[...] / N - mean * mean
    scale = gamma_ref[...] * lax.rsqrt(var + 1e-5)
    shift = beta_ref[...] - mean * scale
    hbuf[pl.ds(0, N), :] = h_ref[...] * scale + shift
    hbuf[pl.ds(N, TB), :] = jnp.zeros((TB, H), jnp.float32)

    hst[...] = jnp.zeros((BP, H), jnp.float32)
    cst[...] = jnp.zeros((BP, H), jnp.float32)

    bias = bih_ref[...] + bhh_ref[...]
    nbv = nbuf[...]

    def block(jb, carry):
        t0 = jb * TB

        def gather(b, c2):
            start = jnp.minimum(cnt_sm[b] + t0 + 1, N)
            xblk[pl.ds(b, 1)] = hbuf[pl.ds(start, TB), :].reshape(1, TB, H)
            return c2

        lax.fori_loop(0, BP, gather, 0)

        for k in range(TB):
            t = t0 + k
            x_t = jnp.where(nbv > t + 1, xblk[:, k, :], 0.0)
            gates = (jnp.dot(x_t, wih_ref[...],
                             preferred_element_type=jnp.float32)
                     + jnp.dot(hst[...], whh_ref[...],
                               preferred_element_type=jnp.float32)
                     + bias)
            i_g = gates[:, 0:H]
            f_g = gates[:, H:2 * H]
            g_g = gates[:, 2 * H:3 * H]
            o_g = gates[:, 3 * H:4 * H]
            c = (jax.nn.sigmoid(f_g) * cst[...]
                 + jax.nn.sigmoid(i_g) * jnp.tanh(g_g))
            h_new = jax.nn.sigmoid(o_g) * jnp.tanh(c)
            active = t < tmax - 1
            hst[...] = jnp.where(active, h_new, hst[...])
            cst[...] = jnp.where(active, c, cst[...])
        return carry

    nblk = (tmax - 1 + TB - 1) // TB
    lax.fori_loop(0, nblk, block, 0)

    s = jnp.sum(hst[...] * fcw_ref[...], axis=1, keepdims=True)
    out_ref[...] = jnp.maximum(s + fcb_ref[...], 0.0)


def kernel(x, edge_index, batch, emb, W_root, W_rel, b_rel, gamma, beta,
           W_ih, W_hh, b_ih, b_hh, fc_W, fc_b):
    src = edge_index[0]
    dst = edge_index[1]

    x_pad = jnp.pad(x, (0, XP - N))
    mesh = plsc.VectorSubcoreMesh(core_axis_name="c", subcore_axis_name="s",
                                  num_cores=NC, num_subcores=NS)
    feat_xp = pl.kernel(
        _emb_gather_kernel,
        out_type=jax.ShapeDtypeStruct((XP, D), jnp.float32),
        mesh=mesh,
        scratch_types=[
            pltpu.VMEM((CH,), jnp.int32),
            pltpu.VMEM((CH,), jnp.int32),
            pltpu.VMEM((CH, D), jnp.float32),
            pltpu.VMEM((CH, D), jnp.float32),
            pltpu.SemaphoreType.DMA,
            pltpu.SemaphoreType.DMA,
        ],
    )(emb, x_pad)

    zeros_tile = jnp.zeros((CH, D), jnp.float32)
    aggr2 = pl.kernel(
        _edge_aggr_kernel,
        out_type=jax.ShapeDtypeStruct((NC * NPAD, D), jnp.float32),
        mesh=mesh,
        scratch_types=[
            pltpu.VMEM((CH,), jnp.int32),
            pltpu.VMEM((CH,), jnp.int32),
            pltpu.VMEM((CH,), jnp.int32),
            pltpu.VMEM((CH,), jnp.int32),
            pltpu.VMEM((CH, D), jnp.float32),
            pltpu.VMEM((CH, D), jnp.float32),
            pltpu.VMEM_SHARED((NPAD, D), jnp.float32),
            pltpu.SemaphoreType.DMA,
            pltpu.SemaphoreType.DMA,
        ],
    )(feat_xp, src, dst, zeros_tile)

    feat = feat_xp[:N]
    p0 = aggr2[:N]
    p1 = aggr2[NPAD:NPAD + N]

    RT = 1000
    NT = N // RT
    h_pre, col_sum, col_sq = pl.pallas_call(
        _graphconv_kernel,
        grid=(NT,),
        in_specs=[
            pl.BlockSpec((RT, D), lambda i: (i, 0)),
            pl.BlockSpec((RT, D), lambda i: (i, 0)),
            pl.BlockSpec((RT, D), lambda i: (i, 0)),
            pl.BlockSpec((D, H), lambda i: (0, 0)),
            pl.BlockSpec((D, H), lambda i: (0, 0)),
            pl.BlockSpec((1, H), lambda i: (0, 0)),
        ],
        out_specs=[
            pl.BlockSpec((RT, H), lambda i: (i, 0)),
            pl.BlockSpec((1, H), lambda i: (0, 0)),
            pl.BlockSpec((1, H), lambda i: (0, 0)),
        ],
        out_shape=[
            jax.ShapeDtypeStruct((N, H), jnp.float32),
            jax.ShapeDtypeStruct((1, H), jnp.float32),
            jax.ShapeDtypeStruct((1, H), jnp.float32),
        ],
    )(p0, p1, feat, W_rel, W_root, b_rel.reshape(1, H))

    batch_pad = jnp.pad(batch, (0, BATCH_PAD - N), constant_values=B)
    out = pl.pallas_call(
        _lstm_kernel,
        out_shape=jax.ShapeDtypeStruct((BP, 1), jnp.float32),
        scratch_shapes=[
            pltpu.VMEM((N + TB, H), jnp.float32),
            pltpu.VMEM((BP, TB, H), jnp.float32),
            pltpu.VMEM((BP, 1), jnp.int32),
            pltpu.VMEM((BP, H), jnp.float32),
            pltpu.VMEM((BP, H), jnp.float32),
            pltpu.SMEM((128,), jnp.int32),
        ],
    )(h_pre, col_sum, col_sq, batch_pad.reshape(BATCH_PAD // CH, CH),
      gamma.reshape(1, H), beta.reshape(1, H),
      W_ih.T, W_hh.T, b_ih.reshape(1, 4 * H), b_hh.reshape(1, 4 * H),
      fc_W.reshape(1, H), fc_b.reshape(1, 1))

    return out[:B, 0]


# speedup vs baseline: 127.8126x; 1.3219x over previous
import functools

import jax
import jax.numpy as jnp
from jax import lax
from jax.experimental import pallas as pl
from jax.experimental.pallas import tpu as pltpu
from jax.experimental.pallas import tpu_sc as plsc

N = 10000
E = 320000
B = 100
D = 128
H = 128
V = 21000

NC = 2
NS = 16
NW = NC * NS
CH = 128

CHG = 320
XP = NW * CHG

ECHUNKS = E // CH
EFULL = ECHUNKS // NW
EREM = ECHUNKS - EFULL * NW

ROWS_PER_TILE = 632
NPAD = NS * ROWS_PER_TILE
_ZW_CHUNKS = [(0, 128), (128, 128), (256, 128), (384, 128), (512, 120)]

BP = 104
TB = 16
BATCH_PAD = (N + CH - 1) // CH * CH


def _emb_gather_kernel(emb_hbm, x_hbm, out_hbm, idx, rows, sem):
    wid = lax.axis_index("s") * NC + lax.axis_index("c")
    base = wid * CHG
    pltpu.sync_copy(x_hbm.at[pl.ds(base, CHG)], idx)
    pltpu.async_copy(emb_hbm.at[idx], rows, sem).wait()
    pltpu.sync_copy(rows, out_hbm.at[pl.ds(base, CHG)])


def _edge_aggr_kernel(feat_hbm, src_hbm, dst_hbm, z_hbm, out_hbm,
                      src0, dst0, src1, dst1, rows0, rows1,
                      aggr_sh, sem0, sem1):
    cid = lax.axis_index("c")
    sid = lax.axis_index("s")
    wid = sid * NC + cid

    r0 = sid * ROWS_PER_TILE
    pltpu.sync_copy(z_hbm, rows0)
    for off, sz in _ZW_CHUNKS:
        pltpu.sync_copy(rows0.at[pl.ds(0, sz)],
                        aggr_sh.at[pl.ds(r0 + off, sz)])
    plsc.subcore_barrier()

    has_rem = wid < EREM
    jlast = EFULL - 1 + jnp.where(has_rem, 1, 0)

    def load(j, s_v, d_v):
        base = (wid + j * NW) * CH
        pltpu.sync_copy(src_hbm.at[pl.ds(base, CH)], s_v)
        pltpu.sync_copy(dst_hbm.at[pl.ds(base, CH)], d_v)

    load(0, src0, dst0)
    pltpu.async_copy(feat_hbm.at[src0], rows0, sem0)

    def drain(s_v, r_v, sem):
        pltpu.make_async_copy(feat_hbm.at[s_v], r_v, sem).wait()

    def body(k, carry):
        load(2 * k + 1, src1, dst1)
        pltpu.async_copy(feat_hbm.at[src1], rows1, sem1)
        drain(src0, rows0, sem0)
        pltpu.sync_copy(rows0, aggr_sh.at[dst0], add=True)
        load(jnp.minimum(2 * k + 2, jlast), src0, dst0)
        pltpu.async_copy(feat_hbm.at[src0], rows0, sem0)
        drain(src1, rows1, sem1)
        pltpu.sync_copy(rows1, aggr_sh.at[dst1], add=True)
        return carry

    lax.fori_loop(0, EFULL // 2, body, 0)

    drain(src0, rows0, sem0)

    @pl.when(has_rem)
    def _():
        pltpu.sync_copy(rows0, aggr_sh.at[dst0], add=True)

    plsc.subcore_barrier()
    for off, sz in _ZW_CHUNKS:
        pltpu.sync_copy(aggr_sh.at[pl.ds(r0 + off, sz)],
                        rows0.at[pl.ds(0, sz)])
        pltpu.sync_copy(rows0.at[pl.ds(0, sz)],
                        out_hbm.at[pl.ds(cid * NPAD + r0 + off, sz)])


def _graphconv_kernel(p0_ref, p1_ref, feat_ref, wrel_ref, wroot_ref,
                      brel_ref, h_ref, sum_ref, sq_ref):
    i = pl.program_id(0)
    a = p0_ref[...] + p1_ref[...]
    h = (jnp.dot(a, wrel_ref[...], preferred_element_type=jnp.float32)
         + jnp.dot(feat_ref[...], wroot_ref[...],
                   preferred_element_type=jnp.float32)
         + brel_ref[...])
    h = jnp.maximum(h, 0.0)
    h_ref[...] = h

    @pl.when(i == 0)
    def _():
        sum_ref[...] = jnp.zeros_like(sum_ref)
        sq_ref[...] = jnp.zeros_like(sq_ref)

    sum_ref[...] += jnp.sum(h, axis=0, keepdims=True)
    sq_ref[...] += jnp.sum(h * h, axis=0, keepdims=True)


def _lstm_kernel(h_ref, sum_ref, sq_ref, batch_ref, gamma_ref, beta_ref,
                 wih_ref, whh_ref, bih_ref, bhh_ref, fcw_ref, fcb_ref,
                 out_ref, hbuf, xblk, nbuf, hst, cst, cnt_sm):
    batch2d = batch_ref[...]

    def count_body(b, carry):
        cnt_sm[b] = jnp.sum(jnp.where(batch2d < b, 1, 0))
        return carry

    lax.fori_loop(0, B + 5, count_body, 0)

    def max_body(b, m):
        return jnp.maximum(m, cnt_sm[b + 1] - cnt_sm[b])

    tmax = lax.fori_loop(0, B, max_body, 0)

    def nb_body(b, carry):
        nb = cnt_sm[b + 1] - cnt_sm[b]
        nbuf[pl.ds(b, 1), :] = jnp.full((1, 1), nb, jnp.int32)
        return carry

    lax.fori_loop(0, BP, nb_body, 0)

    mean = sum_ref[...] / N
    var = sq_ref[...] / N - mean * mean
    scale = gamma_ref[...] * lax.rsqrt(var + 1e-5)
    shift = beta_ref[...] - mean * scale
    hbuf[pl.ds(0, N), :] = h_ref[...] * scale + shift
    hbuf[pl.ds(N, TB), :] = jnp.zeros((TB, H), jnp.float32)

    hst[...] = jnp.zeros((BP, H), jnp.float32)
    cst[...] = jnp.zeros((BP, H), jnp.float32)

    bias = bih_ref[...] + bhh_ref[...]
    nbv = nbuf[...]

    def block(jb, carry):
        t0 = jb * TB

        def gather(b, c2):
            start = jnp.minimum(cnt_sm[b] + t0 + 1, N)
            xblk[pl.ds(b, 1)] = hbuf[pl.ds(start, TB), :].reshape(1, TB, H)
            return c2

        lax.fori_loop(0, BP, gather, 0)

        for k in range(TB):
            t = t0 + k
            x_t = jnp.where(nbv > t + 1, xblk[:, k, :], 0.0)
            gates = (jnp.dot(x_t, wih_ref[...],
                             preferred_element_type=jnp.float32)
                     + jnp.dot(hst[...], whh_ref[...],
                               preferred_element_type=jnp.float32)
                     + bias)
            i_g = gates[:, 0:H]
            f_g = gates[:, H:2 * H]
            g_g = gates[:, 2 * H:3 * H]
            o_g = gates[:, 3 * H:4 * H]
            c = (jax.nn.sigmoid(f_g) * cst[...]
                 + jax.nn.sigmoid(i_g) * jnp.tanh(g_g))
            h_new = jax.nn.sigmoid(o_g) * jnp.tanh(c)
            active = t < tmax - 1
            hst[...] = jnp.where(active, h_new, hst[...])
            cst[...] = jnp.where(active, c, cst[...])
        return carry

    nblk = (tmax - 1 + TB - 1) // TB
    lax.fori_loop(0, nblk, block, 0)

    s = jnp.sum(hst[...] * fcw_ref[...], axis=1, keepdims=True)
    out_ref[...] = jnp.maximum(s + fcb_ref[...], 0.0)


def kernel(x, edge_index, batch, emb, W_root, W_rel, b_rel, gamma, beta,
           W_ih, W_hh, b_ih, b_hh, fc_W, fc_b):
    src = edge_index[0]
    dst = edge_index[1]

    x_pad = jnp.pad(x, (0, XP - N))
    mesh = plsc.VectorSubcoreMesh(core_axis_name="c", subcore_axis_name="s",
                                  num_cores=NC, num_subcores=NS)
    feat_xp = pl.kernel(
        _emb_gather_kernel,
        out_type=jax.ShapeDtypeStruct((XP, D), jnp.float32),
        mesh=mesh,
        scratch_types=[
            pltpu.VMEM((CHG,), jnp.int32),
            pltpu.VMEM((CHG, D), jnp.float32),
            pltpu.SemaphoreType.DMA,
        ],
    )(emb, x_pad)

    zeros_tile = jnp.zeros((CH, D), jnp.float32)
    aggr2 = pl.kernel(
        _edge_aggr_kernel,
        out_type=jax.ShapeDtypeStruct((NC * NPAD, D), jnp.float32),
        mesh=mesh,
        scratch_types=[
            pltpu.VMEM((CH,), jnp.int32),
            pltpu.VMEM((CH,), jnp.int32),
            pltpu.VMEM((CH,), jnp.int32),
            pltpu.VMEM((CH,), jnp.int32),
            pltpu.VMEM((CH, D), jnp.float32),
            pltpu.VMEM((CH, D), jnp.float32),
            pltpu.VMEM_SHARED((NPAD, D), jnp.float32),
            pltpu.SemaphoreType.DMA,
            pltpu.SemaphoreType.DMA,
        ],
    )(feat_xp, src, dst, zeros_tile)

    feat = feat_xp[:N]
    p0 = aggr2[:N]
    p1 = aggr2[NPAD:NPAD + N]

    RT = 1000
    NT = N // RT
    h_pre, col_sum, col_sq = pl.pallas_call(
        _graphconv_kernel,
        grid=(NT,),
        in_specs=[
            pl.BlockSpec((RT, D), lambda i: (i, 0)),
            pl.BlockSpec((RT, D), lambda i: (i, 0)),
            pl.BlockSpec((RT, D), lambda i: (i, 0)),
            pl.BlockSpec((D, H), lambda i: (0, 0)),
            pl.BlockSpec((D, H), lambda i: (0, 0)),
            pl.BlockSpec((1, H), lambda i: (0, 0)),
        ],
        out_specs=[
            pl.BlockSpec((RT, H), lambda i: (i, 0)),
            pl.BlockSpec((1, H), lambda i: (0, 0)),
            pl.BlockSpec((1, H), lambda i: (0, 0)),
        ],
        out_shape=[
            jax.ShapeDtypeStruct((N, H), jnp.float32),
            jax.ShapeDtypeStruct((1, H), jnp.float32),
            jax.ShapeDtypeStruct((1, H), jnp.float32),
        ],
    )(p0, p1, feat, W_rel, W_root, b_rel.reshape(1, H))

    batch_pad = jnp.pad(batch, (0, BATCH_PAD - N), constant_values=B)
    out = pl.pallas_call(
        _lstm_kernel,
        out_shape=jax.ShapeDtypeStruct((BP, 1), jnp.float32),
        scratch_shapes=[
            pltpu.VMEM((N + TB, H), jnp.float32),
            pltpu.VMEM((BP, TB, H), jnp.float32),
            pltpu.VMEM((BP, 1), jnp.int32),
            pltpu.VMEM((BP, H), jnp.float32),
            pltpu.VMEM((BP, H), jnp.float32),
            pltpu.SMEM((128,), jnp.int32),
        ],
    )(h_pre, col_sum, col_sq, batch_pad.reshape(BATCH_PAD // CH, CH),
      gamma.reshape(1, H), beta.reshape(1, H),
      W_ih.T, W_hh.T, b_ih.reshape(1, 4 * H), b_hh.reshape(1, 4 * H),
      fc_W.reshape(1, H), fc_b.reshape(1, 1))

    return out[:B, 0]


# speedup vs baseline: 133.7297x; 1.0463x over previous
import functools

import jax
import jax.numpy as jnp
from jax import lax
from jax.experimental import pallas as pl
from jax.experimental.pallas import tpu as pltpu
from jax.experimental.pallas import tpu_sc as plsc

N = 10000
E = 320000
B = 100
D = 128
H = 128
V = 21000

NC = 2
NS = 16
NW = NC * NS
CH = 128

CHG = 320
XP = NW * CHG

CHE = 160
ECHUNKS = E // CHE
EFULL = ECHUNKS // NW
EREM = ECHUNKS - EFULL * NW

ROWS_PER_TILE = 632
NPAD = NS * ROWS_PER_TILE
_ZW_CHUNKS = [(0, 128), (128, 128), (256, 128), (384, 128), (512, 120)]

BP = 104
TB = 16
BATCH_PAD = (N + CH - 1) // CH * CH


def _emb_gather_kernel(emb_hbm, x_hbm, out_hbm, idx, rows, sem):
    wid = lax.axis_index("s") * NC + lax.axis_index("c")
    base = wid * CHG
    pltpu.sync_copy(x_hbm.at[pl.ds(base, CHG)], idx)
    pltpu.async_copy(emb_hbm.at[idx], rows, sem).wait()
    pltpu.sync_copy(rows, out_hbm.at[pl.ds(base, CHG)])


def _edge_aggr_kernel(feat_hbm, src_hbm, dst_hbm, z_hbm, out_hbm,
                      src0, dst0, src1, dst1, rows0, rows1,
                      aggr_sh, sem0, sem1):
    cid = lax.axis_index("c")
    sid = lax.axis_index("s")
    wid = sid * NC + cid

    r0 = sid * ROWS_PER_TILE
    pltpu.sync_copy(z_hbm, rows0)
    for off, sz in _ZW_CHUNKS:
        pltpu.sync_copy(rows0.at[pl.ds(0, sz)],
                        aggr_sh.at[pl.ds(r0 + off, sz)])
    plsc.subcore_barrier()

    has_rem = wid < EREM
    jlast = EFULL - 1 + jnp.where(has_rem, 1, 0)

    def load(j, s_v, d_v):
        base = (wid + j * NW) * CHE
        pltpu.sync_copy(src_hbm.at[pl.ds(base, CHE)], s_v)
        pltpu.sync_copy(dst_hbm.at[pl.ds(base, CHE)], d_v)

    load(0, src0, dst0)
    pltpu.async_copy(feat_hbm.at[src0], rows0, sem0)

    def drain(s_v, r_v, sem):
        pltpu.make_async_copy(feat_hbm.at[s_v], r_v, sem).wait()

    def body(k, carry):
        load(2 * k + 1, src1, dst1)
        pltpu.async_copy(feat_hbm.at[src1], rows1, sem1)
        drain(src0, rows0, sem0)
        pltpu.sync_copy(rows0, aggr_sh.at[dst0], add=True)
        load(jnp.minimum(2 * k + 2, jlast), src0, dst0)
        pltpu.async_copy(feat_hbm.at[src0], rows0, sem0)
        drain(src1, rows1, sem1)
        pltpu.sync_copy(rows1, aggr_sh.at[dst1], add=True)
        return carry

    lax.fori_loop(0, EFULL // 2, body, 0)

    drain(src0, rows0, sem0)

    @pl.when(has_rem)
    def _():
        pltpu.sync_copy(rows0, aggr_sh.at[dst0], add=True)

    plsc.subcore_barrier()
    for off, sz in _ZW_CHUNKS:
        pltpu.sync_copy(aggr_sh.at[pl.ds(r0 + off, sz)],
                        rows0.at[pl.ds(0, sz)])
        pltpu.sync_copy(rows0.at[pl.ds(0, sz)],
                        out_hbm.at[pl.ds(cid * NPAD + r0 + off, sz)])


def _graphconv_kernel(p0_ref, p1_ref, feat_ref, wrel_ref, wroot_ref,
                      brel_ref, h_ref, sum_ref, sq_ref):
    i = pl.program_id(0)
    a = p0_ref[...] + p1_ref[...]
    h = (jnp.dot(a, wrel_ref[...], preferred_element_type=jnp.float32)
         + jnp.dot(feat_ref[...], wroot_ref[...],
                   preferred_element_type=jnp.float32)
         + brel_ref[...])
    h = jnp.maximum(h, 0.0)
    h_ref[...] = h

    @pl.when(i == 0)
    def _():
        sum_ref[...] = jnp.zeros_like(sum_ref)
        sq_ref[...] = jnp.zeros_like(sq_ref)

    sum_ref[...] += jnp.sum(h, axis=0, keepdims=True)
    sq_ref[...] += jnp.sum(h * h, axis=0, keepdims=True)


def _lstm_kernel(h_ref, sum_ref, sq_ref, batch_ref, gamma_ref, beta_ref,
                 wih_ref, whh_ref, bih_ref, bhh_ref, fcw_ref, fcb_ref,
                 out_ref, hbuf, xblk, nbuf, hst, cst, cnt_sm):
    batch2d = batch_ref[...]

    def count_body(b, carry):
        cnt_sm[b] = jnp.sum(jnp.where(batch2d < b, 1, 0))
        return carry

    lax.fori_loop(0, B + 5, count_body, 0)

    def max_body(b, m):
        return jnp.maximum(m, cnt_sm[b + 1] - cnt_sm[b])

    tmax = lax.fori_loop(0, B, max_body, 0)

    def nb_body(b, carry):
        nb = cnt_sm[b + 1] - cnt_sm[b]
        nbuf[pl.ds(b, 1), :] = jnp.full((1, 1), nb, jnp.int32)
        return carry

    lax.fori_loop(0, BP, nb_body, 0)

    mean = sum_ref[...] / N
    var = sq_ref[...] / N - mean * mean
    scale = gamma_ref[...] * lax.rsqrt(var + 1e-5)
    shift = beta_ref[...] - mean * scale
    hbuf[pl.ds(0, N), :] = h_ref[...] * scale + shift
    hbuf[pl.ds(N, TB), :] = jnp.zeros((TB, H), jnp.float32)

    hst[...] = jnp.zeros((BP, H), jnp.float32)
    cst[...] = jnp.zeros((BP, H), jnp.float32)

    bias = bih_ref[...] + bhh_ref[...]
    nbv = nbuf[...]

    def block(jb, carry):
        t0 = jb * TB

        def gather(b, c2):
            start = jnp.minimum(cnt_sm[b] + t0 + 1, N)
            xblk[pl.ds(b, 1)] = hbuf[pl.ds(start, TB), :].reshape(1, TB, H)
            return c2

        lax.fori_loop(0, BP, gather, 0)

        for k in range(TB):
            t = t0 + k
            x_t = jnp.where(nbv > t + 1, xblk[:, k, :], 0.0)
            gates = (jnp.dot(x_t, wih_ref[...],
                             preferred_element_type=jnp.float32)
                     + jnp.dot(hst[...], whh_ref[...],
                               preferred_element_type=jnp.float32)
                     + bias)
            i_g = gates[:, 0:H]
            f_g = gates[:, H:2 * H]
            g_g = gates[:, 2 * H:3 * H]
            o_g = gates[:, 3 * H:4 * H]
            c = (jax.nn.sigmoid(f_g) * cst[...]
                 + jax.nn.sigmoid(i_g) * jnp.tanh(g_g))
            h_new = jax.nn.sigmoid(o_g) * jnp.tanh(c)
            active = t < tmax - 1
            hst[...] = jnp.where(active, h_new, hst[...])
            cst[...] = jnp.where(active, c, cst[...])
        return carry

    nblk = (tmax - 1 + TB - 1) // TB
    lax.fori_loop(0, nblk, block, 0)

    s = jnp.sum(hst[...] * fcw_ref[...], axis=1, keepdims=True)
    out_ref[...] = jnp.maximum(s + fcb_ref[...], 0.0)


def kernel(x, edge_index, batch, emb, W_root, W_rel, b_rel, gamma, beta,
           W_ih, W_hh, b_ih, b_hh, fc_W, fc_b):
    src = edge_index[0]
    dst = edge_index[1]

    x_pad = jnp.pad(x, (0, XP - N))
    mesh = plsc.VectorSubcoreMesh(core_axis_name="c", subcore_axis_name="s",
                                  num_cores=NC, num_subcores=NS)
    feat_xp = pl.kernel(
        _emb_gather_kernel,
        out_type=jax.ShapeDtypeStruct((XP, D), jnp.float32),
        mesh=mesh,
        scratch_types=[
            pltpu.VMEM((CHG,), jnp.int32),
            pltpu.VMEM((CHG, D), jnp.float32),
            pltpu.SemaphoreType.DMA,
        ],
    )(emb, x_pad)

    zeros_tile = jnp.zeros((CHE, D), jnp.float32)
    aggr2 = pl.kernel(
        _edge_aggr_kernel,
        out_type=jax.ShapeDtypeStruct((NC * NPAD, D), jnp.float32),
        mesh=mesh,
        scratch_types=[
            pltpu.VMEM((CHE,), jnp.int32),
            pltpu.VMEM((CHE,), jnp.int32),
            pltpu.VMEM((CHE,), jnp.int32),
            pltpu.VMEM((CHE,), jnp.int32),
            pltpu.VMEM((CHE, D), jnp.float32),
            pltpu.VMEM((CHE, D), jnp.float32),
            pltpu.VMEM_SHARED((NPAD, D), jnp.float32),
            pltpu.SemaphoreType.DMA,
            pltpu.SemaphoreType.DMA,
        ],
    )(feat_xp, src, dst, zeros_tile)

    feat = feat_xp[:N]
    p0 = aggr2[:N]
    p1 = aggr2[NPAD:NPAD + N]

    RT = 1000
    NT = N // RT
    h_pre, col_sum, col_sq = pl.pallas_call(
        _graphconv_kernel,
        grid=(NT,),
        in_specs=[
            pl.BlockSpec((RT, D), lambda i: (i, 0)),
            pl.BlockSpec((RT, D), lambda i: (i, 0)),
            pl.BlockSpec((RT, D), lambda i: (i, 0)),
            pl.BlockSpec((D, H), lambda i: (0, 0)),
            pl.BlockSpec((D, H), lambda i: (0, 0)),
            pl.BlockSpec((1, H), lambda i: (0, 0)),
        ],
        out_specs=[
            pl.BlockSpec((RT, H), lambda i: (i, 0)),
            pl.BlockSpec((1, H), lambda i: (0, 0)),
            pl.BlockSpec((1, H), lambda i: (0, 0)),
        ],
        out_shape=[
            jax.ShapeDtypeStruct((N, H), jnp.float32),
            jax.ShapeDtypeStruct((1, H), jnp.float32),
            jax.ShapeDtypeStruct((1, H), jnp.float32),
        ],
    )(p0, p1, feat, W_rel, W_root, b_rel.reshape(1, H))

    batch_pad = jnp.pad(batch, (0, BATCH_PAD - N), constant_values=B)
    out = pl.pallas_call(
        _lstm_kernel,
        out_shape=jax.ShapeDtypeStruct((BP, 1), jnp.float32),
        scratch_shapes=[
            pltpu.VMEM((N + TB, H), jnp.float32),
            pltpu.VMEM((BP, TB, H), jnp.float32),
            pltpu.VMEM((BP, 1), jnp.int32),
            pltpu.VMEM((BP, H), jnp.float32),
            pltpu.VMEM((BP, H), jnp.float32),
            pltpu.SMEM((128,), jnp.int32),
        ],
    )(h_pre, col_sum, col_sq, batch_pad.reshape(BATCH_PAD // CH, CH),
      gamma.reshape(1, H), beta.reshape(1, H),
      W_ih.T, W_hh.T, b_ih.reshape(1, 4 * H), b_hh.reshape(1, 4 * H),
      fc_W.reshape(1, H), fc_b.reshape(1, 1))

    return out[:B, 0]


# speedup vs baseline: 136.5632x; 1.0212x over previous
import functools

import jax
import jax.numpy as jnp
from jax import lax
from jax.experimental import pallas as pl
from jax.experimental.pallas import tpu as pltpu
from jax.experimental.pallas import tpu_sc as plsc

N = 10000
E = 320000
B = 100
D = 128
H = 128
V = 21000

NC = 2
NS = 16
NW = NC * NS
CH = 128

CHG = 320
XP = NW * CHG

CHE = 160
ECHUNKS = E // CHE
EFULL = ECHUNKS // NW
EREM = ECHUNKS - EFULL * NW

ROWS_PER_TILE = 632
NPAD = NS * ROWS_PER_TILE
_ZW_CHUNKS = [(0, 128), (128, 128), (256, 128), (384, 128), (512, 120)]

BP = 104
TB = 16
BATCH_PAD = (N + CH - 1) // CH * CH


def _emb_gather_kernel(emb_hbm, x_hbm, out_hbm, idx, rows, sem):
    wid = lax.axis_index("s") * NC + lax.axis_index("c")
    base = wid * CHG
    pltpu.sync_copy(x_hbm.at[pl.ds(base, CHG)], idx)
    pltpu.async_copy(emb_hbm.at[idx], rows, sem).wait()
    pltpu.sync_copy(rows, out_hbm.at[pl.ds(base, CHG)])


def _edge_aggr_kernel(feat_hbm, src_hbm, dst_hbm, z_hbm, out_hbm,
                      src0, dst0, src1, dst1, rows0, rows1,
                      aggr_sh, sem0, sem1):
    cid = lax.axis_index("c")
    sid = lax.axis_index("s")
    wid = sid * NC + cid

    r0 = sid * ROWS_PER_TILE
    pltpu.sync_copy(z_hbm, rows0)
    for off, sz in _ZW_CHUNKS:
        pltpu.sync_copy(rows0.at[pl.ds(0, sz)],
                        aggr_sh.at[pl.ds(r0 + off, sz)])
    plsc.subcore_barrier()

    has_rem = wid < EREM
    jlast = EFULL - 1 + jnp.where(has_rem, 1, 0)

    def load(j, s_v, d_v):
        base = (wid + j * NW) * CHE
        pltpu.sync_copy(src_hbm.at[pl.ds(base, CHE)], s_v)
        pltpu.sync_copy(dst_hbm.at[pl.ds(base, CHE)], d_v)

    load(0, src0, dst0)
    pltpu.async_copy(feat_hbm.at[src0], rows0, sem0)

    def drain(s_v, r_v, sem):
        pltpu.make_async_copy(feat_hbm.at[s_v], r_v, sem).wait()

    def body(k, carry):
        load(2 * k + 1, src1, dst1)
        pltpu.async_copy(feat_hbm.at[src1], rows1, sem1)
        drain(src0, rows0, sem0)
        pltpu.sync_copy(rows0, aggr_sh.at[dst0], add=True)
        load(jnp.minimum(2 * k + 2, jlast), src0, dst0)
        pltpu.async_copy(feat_hbm.at[src0], rows0, sem0)
        drain(src1, rows1, sem1)
        pltpu.sync_copy(rows1, aggr_sh.at[dst1], add=True)
        return carry

    lax.fori_loop(0, EFULL // 2, body, 0)

    drain(src0, rows0, sem0)

    @pl.when(has_rem)
    def _():
        pltpu.sync_copy(rows0, aggr_sh.at[dst0], add=True)

    plsc.subcore_barrier()
    for off, sz in _ZW_CHUNKS:
        pltpu.sync_copy(aggr_sh.at[pl.ds(r0 + off, sz)],
                        rows0.at[pl.ds(0, sz)])
        pltpu.sync_copy(rows0.at[pl.ds(0, sz)],
                        out_hbm.at[pl.ds(cid * NPAD + r0 + off, sz)])


RT = 1000
NT = N // RT


def _gc_lstm_kernel(p0_ref, p1_ref, feat_ref, wrel_ref, wroot_ref, brel_ref,
                    batch_ref, gamma_ref, beta_ref,
                    wih_ref, whh_ref, bih_ref, bhh_ref, fcw_ref, fcb_ref,
                    out_ref, hbuf, xblk, nbuf, hst, cst, cnt_sm,
                    sum_sc, sq_sc):
    i = pl.program_id(0)

    @pl.when(i == 0)
    def _():
        sum_sc[...] = jnp.zeros_like(sum_sc)
        sq_sc[...] = jnp.zeros_like(sq_sc)

    @pl.when(i < NT)
    def _():
        a = p0_ref[...] + p1_ref[...]
        h = (jnp.dot(a, wrel_ref[...], preferred_element_type=jnp.float32)
             + jnp.dot(feat_ref[...], wroot_ref[...],
                       preferred_element_type=jnp.float32)
             + brel_ref[...])
        h = jnp.maximum(h, 0.0)
        hbuf[pl.ds(i * RT, RT), :] = h
        sum_sc[...] += jnp.sum(h, axis=0, keepdims=True)
        sq_sc[...] += jnp.sum(h * h, axis=0, keepdims=True)

    @pl.when(i == NT)
    def _():
        _lstm_tail(batch_ref, gamma_ref, beta_ref, wih_ref, whh_ref,
                   bih_ref, bhh_ref, fcw_ref, fcb_ref, out_ref,
                   hbuf, xblk, nbuf, hst, cst, cnt_sm, sum_sc, sq_sc)


def _lstm_tail(batch_ref, gamma_ref, beta_ref,
               wih_ref, whh_ref, bih_ref, bhh_ref, fcw_ref, fcb_ref,
               out_ref, hbuf, xblk, nbuf, hst, cst, cnt_sm, sum_ref, sq_ref):
    batch2d = batch_ref[...]

    def count_body(b, carry):
        cnt_sm[b] = jnp.sum(jnp.where(batch2d < b, 1, 0))
        return carry

    lax.fori_loop(0, B + 5, count_body, 0)

    def max_body(b, m):
        return jnp.maximum(m, cnt_sm[b + 1] - cnt_sm[b])

    tmax = lax.fori_loop(0, B, max_body, 0)

    def nb_body(b, carry):
        nb = cnt_sm[b + 1] - cnt_sm[b]
        nbuf[pl.ds(b, 1), :] = jnp.full((1, 1), nb, jnp.int32)
        return carry

    lax.fori_loop(0, BP, nb_body, 0)

    mean = sum_ref[...] / N
    var = sq_ref[...] / N - mean * mean
    scale = gamma_ref[...] * lax.rsqrt(var + 1e-5)
    shift = beta_ref[...] - mean * scale
    hbuf[pl.ds(0, N), :] = hbuf[pl.ds(0, N), :] * scale + shift
    hbuf[pl.ds(N, TB), :] = jnp.zeros((TB, H), jnp.float32)

    hst[...] = jnp.zeros((BP, H), jnp.float32)
    cst[...] = jnp.zeros((BP, H), jnp.float32)

    bias = bih_ref[...] + bhh_ref[...]
    nbv = nbuf[...]

    def block(jb, carry):
        t0 = jb * TB

        def gather(b, c2):
            start = jnp.minimum(cnt_sm[b] + t0 + 1, N)
            xblk[pl.ds(b, 1)] = hbuf[pl.ds(start, TB), :].reshape(1, TB, H)
            return c2

        lax.fori_loop(0, BP, gather, 0)

        for k in range(TB):
            t = t0 + k
            x_t = jnp.where(nbv > t + 1, xblk[:, k, :], 0.0)
            gates = (jnp.dot(x_t, wih_ref[...],
                             preferred_element_type=jnp.float32)
                     + jnp.dot(hst[...], whh_ref[...],
                               preferred_element_type=jnp.float32)
                     + bias)
            i_g = gates[:, 0:H]
            f_g = gates[:, H:2 * H]
            g_g = gates[:, 2 * H:3 * H]
            o_g = gates[:, 3 * H:4 * H]
            c = (jax.nn.sigmoid(f_g) * cst[...]
                 + jax.nn.sigmoid(i_g) * jnp.tanh(g_g))
            h_new = jax.nn.sigmoid(o_g) * jnp.tanh(c)
            active = t < tmax - 1
            hst[...] = jnp.where(active, h_new, hst[...])
            cst[...] = jnp.where(active, c, cst[...])
        return carry

    nblk = (tmax - 1 + TB - 1) // TB
    lax.fori_loop(0, nblk, block, 0)

    s = jnp.sum(hst[...] * fcw_ref[...], axis=1, keepdims=True)
    out_ref[...] = jnp.maximum(s + fcb_ref[...], 0.0)


def kernel(x, edge_index, batch, emb, W_root, W_rel, b_rel, gamma, beta,
           W_ih, W_hh, b_ih, b_hh, fc_W, fc_b):
    src = edge_index[0]
    dst = edge_index[1]

    x_pad = jnp.pad(x, (0, XP - N))
    mesh = plsc.VectorSubcoreMesh(core_axis_name="c", subcore_axis_name="s",
                                  num_cores=NC, num_subcores=NS)
    feat_xp = pl.kernel(
        _emb_gather_kernel,
        out_type=jax.ShapeDtypeStruct((XP, D), jnp.float32),
        mesh=mesh,
        scratch_types=[
            pltpu.VMEM((CHG,), jnp.int32),
            pltpu.VMEM((CHG, D), jnp.float32),
            pltpu.SemaphoreType.DMA,
        ],
    )(emb, x_pad)

    zeros_tile = jnp.zeros((CHE, D), jnp.float32)
    aggr2 = pl.kernel(
        _edge_aggr_kernel,
        out_type=jax.ShapeDtypeStruct((NC * NPAD, D), jnp.float32),
        mesh=mesh,
        scratch_types=[
            pltpu.VMEM((CHE,), jnp.int32),
            pltpu.VMEM((CHE,), jnp.int32),
            pltpu.VMEM((CHE,), jnp.int32),
            pltpu.VMEM((CHE,), jnp.int32),
            pltpu.VMEM((CHE, D), jnp.float32),
            pltpu.VMEM((CHE, D), jnp.float32),
            pltpu.VMEM_SHARED((NPAD, D), jnp.float32),
            pltpu.SemaphoreType.DMA,
            pltpu.SemaphoreType.DMA,
        ],
    )(feat_xp, src, dst, zeros_tile)

    feat = feat_xp[:N]
    p0 = aggr2[:N]
    p1 = aggr2[NPAD:NPAD + N]

    batch_pad = jnp.pad(batch, (0, BATCH_PAD - N), constant_values=B)
    last = NT - 1
    out = pl.pallas_call(
        _gc_lstm_kernel,
        grid=(NT + 1,),
        in_specs=[
            pl.BlockSpec((RT, D), lambda i: (jnp.minimum(i, last), 0)),
            pl.BlockSpec((RT, D), lambda i: (jnp.minimum(i, last), 0)),
            pl.BlockSpec((RT, D), lambda i: (jnp.minimum(i, last), 0)),
            pl.BlockSpec((D, H), lambda i: (0, 0)),
            pl.BlockSpec((D, H), lambda i: (0, 0)),
            pl.BlockSpec((1, H), lambda i: (0, 0)),
            pl.BlockSpec((BATCH_PAD // CH, CH), lambda i: (0, 0)),
            pl.BlockSpec((1, H), lambda i: (0, 0)),
            pl.BlockSpec((1, H), lambda i: (0, 0)),
            pl.BlockSpec((H, 4 * H), lambda i: (0, 0)),
            pl.BlockSpec((H, 4 * H), lambda i: (0, 0)),
            pl.BlockSpec((1, 4 * H), lambda i: (0, 0)),
            pl.BlockSpec((1, 4 * H), lambda i: (0, 0)),
            pl.BlockSpec((1, H), lambda i: (0, 0)),
            pl.BlockSpec((1, 1), lambda i: (0, 0)),
        ],
        out_specs=pl.BlockSpec((BP, 1), lambda i: (0, 0)),
        out_shape=jax.ShapeDtypeStruct((BP, 1), jnp.float32),
        scratch_shapes=[
            pltpu.VMEM((N + TB, H), jnp.float32),
            pltpu.VMEM((BP, TB, H), jnp.float32),
            pltpu.VMEM((BP, 1), jnp.int32),
            pltpu.VMEM((BP, H), jnp.float32),
            pltpu.VMEM((BP, H), jnp.float32),
            pltpu.SMEM((128,), jnp.int32),
            pltpu.VMEM((1, H), jnp.float32),
            pltpu.VMEM((1, H), jnp.float32),
        ],
    )(p0, p1, feat, W_rel, W_root, b_rel.reshape(1, H),
      batch_pad.reshape(BATCH_PAD // CH, CH),
      gamma.reshape(1, H), beta.reshape(1, H),
      W_ih.T, W_hh.T, b_ih.reshape(1, 4 * H), b_hh.reshape(1, 4 * H),
      fc_W.reshape(1, H), fc_b.reshape(1, 1))

    return out[:B, 0]
